# baseline (speedup 1.0000x reference)
import jax
import jax.numpy as jnp
from jax import lax
from jax.experimental import pallas as pl
from jax.experimental.pallas import tpu as pltpu
from jax.experimental.pallas import tpu_sc as plsc

N = 50000
E = 800000
D = 96
L = 16
NCHUNK = D // L
NSUB = 16
NCORE = 2
NPAIR = NCHUNK // NCORE
W = 128
NWIN = 392
EPT = NWIN * W
EPAD = EPT * NSUB
NPAD = 52000
ROWS_PER_SUB = N // NSUB
ZROWS = 125
BN = 2000
NBLK = NPAD // BN


def _node_mm_body(x_ref, w_ref, b_ref, o_ref):
    o_ref[...] = (
        jnp.dot(x_ref[...], w_ref[...], preferred_element_type=jnp.float32)
        + b_ref[...]
    )


def _edge_mm_body(a_ref, w_ref, o_ref):
    o_ref[...] = jnp.dot(a_ref[...], w_ref[...], preferred_element_type=jnp.float32)


def _table_mm_body(x_ref, wd_ref, we_ref, bd_ref, be_ref, dxt_ref, ebt_ref):
    i = pl.program_id(1)

    @pl.when(i < NBLK - 1)
    def _():
        dxt_ref[...] = (
            jnp.dot(x_ref[...], wd_ref[0], preferred_element_type=jnp.float32)
            + bd_ref[0]
        )
        ebt_ref[...] = (
            jnp.dot(x_ref[...], we_ref[0], preferred_element_type=jnp.float32)
            + be_ref[0]
        )

    @pl.when(i == NBLK - 1)
    def _():
        dxt_ref[...] = jnp.full_like(dxt_ref, 1e30)
        ebt_ref[...] = jnp.zeros_like(ebt_ref)


def _final_body(ax_ref, x_ref, g_ref, be_ref, a0, a1, a2, a3, a4, a5,
                o_ref, stat_ref):
    p = pl.program_id(0)
    accs = (a0, a1, a2, a3, a4, a5)
    num = jnp.concatenate([a[...][:, 0:L] for a in accs], axis=1)
    den = jnp.concatenate([a[...][:, L:2 * L] for a in accs], axis=1)
    h = ax_ref[...] + num / (den + 1e-6)

    @pl.when(p == 0)
    def _():
        @pl.when(pl.program_id(1) == 0)
        def _():
            stat_ref[...] = jnp.zeros_like(stat_ref)

        stat_ref[0:1, 0:D] += jnp.sum(h, axis=0, keepdims=True)
        stat_ref[1:2, 0:D] += jnp.sum(h * h, axis=0, keepdims=True)

    @pl.when(p == 1)
    def _():
        mean = stat_ref[0:1, 0:D] / N
        var = stat_ref[1:2, 0:D] / N - mean * mean
        bn = g_ref[...] * (h - mean) / jnp.sqrt(var + 1e-5) + be_ref[...]
        o_ref[...] = x_ref[...] + jnp.maximum(bn, 0.0)


def _sc_body(dxt, ebt, ce, src_idx, dst_idx, out,
             acc, dst_raw, dst_adj, src_adj, scidx, dbuf, ebbuf, cbuf, updbuf,
             sem_i0, sem_i1, sem_g0, sem_g1, sem_s):
    core = lax.axis_index("c")
    sid = lax.axis_index("s")
    semi = (sem_i0, sem_i1)
    semg = (sem_g0, sem_g1)
    cbase = core * NPAD

    def _z(r, carry):
        updbuf[0, r, 0:L] = jnp.zeros((L,), jnp.float32)
        updbuf[0, r, L:2 * L] = jnp.zeros((L,), jnp.float32)
        return carry
    lax.fori_loop(0, ZROWS, _z, 0)

    def _zero(k, carry):
        pltpu.sync_copy(updbuf.at[0, pl.ds(0, ZROWS)],
                        acc.at[pl.ds(sid * ROWS_PER_SUB + k * ZROWS, ZROWS)])
        return carry
    lax.fori_loop(0, ROWS_PER_SUB // ZROWS, _zero, 0)
    plsc.subcore_barrier()

    def issue_i(w, par):
        base = sid * EPT + w * W
        pltpu.async_copy(dst_idx.at[pl.ds(base, W)], dst_raw.at[par], semi[par])
        pltpu.async_copy(src_idx.at[pl.ds(base, W)], src_adj.at[par], semi[par])

    def wait_i(par):
        pltpu.make_async_copy(dst_idx.at[pl.ds(0, W)], dst_raw.at[par], semi[par]).wait()
        pltpu.make_async_copy(src_idx.at[pl.ds(0, W)], src_adj.at[par], semi[par]).wait()

    def do_adj(par):
        for k in range(W // L):
            sl = pl.ds(k * L, L)
            dst_adj[par, sl] = dst_raw[par, sl] + cbase
            src_adj[par, sl] = src_adj[par, sl] + cbase

    def do_clamp(par):
        for k in range(W // L):
            sl = pl.ds(k * L, L)
            scidx[par, sl] = jnp.minimum(dst_raw[par, sl], N - 1)

    def issue_g(w, par):
        base = sid * EPT + w * W
        bce = jnp.minimum(base, E - W)
        pltpu.async_copy(dxt.at[dst_adj.at[par]], dbuf.at[par], semg[par])
        pltpu.async_copy(ebt.at[src_adj.at[par]], ebbuf.at[par], semg[par])
        pltpu.async_copy(ce.at[pl.ds(bce, W), pl.ds(core * L, L)], cbuf.at[par],
                         semg[par])

    def wait_g(par):
        pltpu.make_async_copy(dxt.at[pl.ds(0, W)], dbuf.at[par], semg[par]).wait()
        pltpu.make_async_copy(ebt.at[pl.ds(0, W)], ebbuf.at[par], semg[par]).wait()
        pltpu.make_async_copy(ce.at[pl.ds(0, W), pl.ds(0, L)], cbuf.at[par], semg[par]).wait()

    def wait_s(par):
        pltpu.make_async_copy(ebt.at[pl.ds(0, W)], updbuf.at[par], sem_s).wait()

    def compute(par):
        @plsc.parallel_loop(0, W, unroll=8)
        def _(e):
            ev = dbuf[par, e, :] + ebbuf[par, e, 0:L] + cbuf[par, e, :]
            sig = 1.0 / (1.0 + jnp.exp(ev))
            updbuf[par, e, 0:L] = sig * ebbuf[par, e, L:2 * L]
            updbuf[par, e, L:2 * L] = sig

    def run_window(w, t, par, drain_guard, prep_guard, issue_guard):
        q = 1 - par
        wait_g(par)

        def _prep():
            wait_i(q)
            do_adj(q)
            issue_g(w + 1, q)
        if prep_guard:
            pl.when(t < NWIN // 2 - 1)(_prep)
        else:
            _prep()
        compute(par)

        def _drain():
            wait_s(q)
        if drain_guard:
            pl.when(t > 0)(_drain)
        else:
            _drain()
        do_clamp(par)
        pltpu.async_copy(updbuf.at[par], acc.at[scidx.at[par]], sem_s, add=True)

        def _issue_next():
            issue_i(w + 2, par)
        pl.when(t < NWIN // 2 - 1)(_issue_next)

    issue_i(0, 0)
    issue_i(1, 1)
    wait_i(0)
    do_adj(0)
    issue_g(0, 0)

    def _witer(t, carry):
        run_window(2 * t, t, 0, True, False, True)
        run_window(2 * t + 1, t, 1, False, True, True)
        return carry
    lax.fori_loop(0, NWIN // 2, _witer, 0)
    wait_s(0)
    plsc.subcore_barrier()

    pltpu.sync_copy(
        acc.at[pl.ds(sid * ROWS_PER_SUB, ROWS_PER_SUB)],
        out.at[pl.ds(core * N + sid * ROWS_PER_SUB, ROWS_PER_SUB)],
    )
    plsc.subcore_barrier()


def kernel(x, edge_index, edge_attr, W_A, b_A, W_B, b_B, W_C, b_C, W_D, b_D,
           W_E, b_E, gamma_x, beta_x, gamma_e, beta_e):
    f32 = jnp.float32

    ax = pl.pallas_call(
        _node_mm_body,
        grid=(N // BN,),
        in_specs=[
            pl.BlockSpec((BN, D), lambda i: (i, 0)),
            pl.BlockSpec((D, D), lambda i: (0, 0)),
            pl.BlockSpec((1, D), lambda i: (0, 0)),
        ],
        out_specs=pl.BlockSpec((BN, D), lambda i: (i, 0)),
        out_shape=jax.ShapeDtypeStruct((N, D), f32),
    )(x, W_A, b_A.reshape(1, D))

    npd = EPAD - E
    ar = jnp.arange(npd, dtype=jnp.int32)
    src_pad = jnp.concatenate([edge_index[0], ar % 64])
    dst_pad = jnp.concatenate([edge_index[1], N + (ar % 64)])

    wd4 = (-W_D).reshape(D, NCHUNK, L).transpose(1, 0, 2)
    we4 = jnp.concatenate(
        [(-W_E).reshape(D, NCHUNK, L), W_B.reshape(D, NCHUNK, L)], axis=2
    ).transpose(1, 0, 2)
    bd4 = (-(b_D + b_E + b_C)).reshape(NCHUNK, 1, L)
    be4 = jnp.concatenate(
        [jnp.zeros((NCHUNK, 1, L), f32), b_B.reshape(NCHUNK, 1, L)], axis=2)
    wc = -W_C

    mesh = plsc.VectorSubcoreMesh(core_axis_name="c", subcore_axis_name="s")
    BE = 2000
    accs = []
    for j in range(NPAIR):
        dxt, ebt = pl.pallas_call(
            _table_mm_body,
            grid=(NCORE, NBLK),
            in_specs=[
                pl.BlockSpec((BN, D), lambda c, i: (jnp.minimum(i, NBLK - 2), 0)),
                pl.BlockSpec((1, D, L), lambda c, i: (c, 0, 0)),
                pl.BlockSpec((1, D, 2 * L), lambda c, i: (c, 0, 0)),
                pl.BlockSpec((1, 1, L), lambda c, i: (c, 0, 0)),
                pl.BlockSpec((1, 1, 2 * L), lambda c, i: (c, 0, 0)),
            ],
            out_specs=[
                pl.BlockSpec((BN, L), lambda c, i: (c * NBLK + i, 0)),
                pl.BlockSpec((BN, 2 * L), lambda c, i: (c * NBLK + i, 0)),
            ],
            out_shape=[
                jax.ShapeDtypeStruct((NCORE * NPAD, L), f32),
                jax.ShapeDtypeStruct((NCORE * NPAD, 2 * L), f32),
            ],
        )(x, wd4[2 * j:2 * j + 2], we4[2 * j:2 * j + 2],
          bd4[2 * j:2 * j + 2], be4[2 * j:2 * j + 2])

        ce = pl.pallas_call(
            _edge_mm_body,
            grid=(E // BE,),
            in_specs=[
                pl.BlockSpec((BE, D), lambda i: (i, 0)),
                pl.BlockSpec((D, NCORE * L), lambda i: (0, 0)),
            ],
            out_specs=pl.BlockSpec((BE, NCORE * L), lambda i: (i, 0)),
            out_shape=jax.ShapeDtypeStruct((E, NCORE * L), f32),
        )(edge_attr, wc[:, 2 * j * L:(2 * j + 2) * L])

        accs.append(pl.kernel(
            _sc_body,
            out_type=jax.ShapeDtypeStruct((NCORE * N, 2 * L), f32),
            mesh=mesh,
            compiler_params=pltpu.CompilerParams(
                use_tc_tiling_on_sc=False, internal_scratch_in_bytes=256 * 1024),
            scratch_types=[
                pltpu.VMEM_SHARED((N, 2 * L), f32),
                pltpu.VMEM((2, W), jnp.int32),
                pltpu.VMEM((2, W), jnp.int32),
                pltpu.VMEM((2, W), jnp.int32),
                pltpu.VMEM((2, W), jnp.int32),
                pltpu.VMEM((2, W, L), f32),
                pltpu.VMEM((2, W, 2 * L), f32),
                pltpu.VMEM((2, W, L), f32),
                pltpu.VMEM((2, W, 2 * L), f32),
                pltpu.SemaphoreType.DMA,
                pltpu.SemaphoreType.DMA,
                pltpu.SemaphoreType.DMA,
                pltpu.SemaphoreType.DMA,
                pltpu.SemaphoreType.DMA,
            ],
        )(dxt, ebt, ce, src_pad, dst_pad))

    def _accspec(c):
        return pl.BlockSpec((BN, 2 * L), lambda p, i, c=c: (c * (N // BN) + i, 0))

    x_out = pl.pallas_call(
        _final_body,
        grid=(2, N // BN),
        in_specs=[
            pl.BlockSpec((BN, D), lambda p, i: (i, 0)),
            pl.BlockSpec((BN, D), lambda p, i: (i, 0)),
            pl.BlockSpec((1, D), lambda p, i: (0, 0)),
            pl.BlockSpec((1, D), lambda p, i: (0, 0)),
        ] + [_accspec(c) for _ in range(NPAIR) for c in range(NCORE)],
        out_specs=pl.BlockSpec((BN, D), lambda p, i: (i, 0)),
        out_shape=jax.ShapeDtypeStruct((N, D), f32),
        scratch_shapes=[pltpu.VMEM((8, 128), f32)],
    )(ax, x, gamma_x.reshape(1, D), beta_x.reshape(1, D),
      accs[0], accs[0], accs[1], accs[1], accs[2], accs[2])

    return x_out


# speedup vs baseline: 1.3604x; 1.3604x over previous
import functools

import jax
import jax.numpy as jnp
from jax import lax
from jax.experimental import pallas as pl
from jax.experimental.pallas import tpu as pltpu
from jax.experimental.pallas import tpu_sc as plsc

N = 50000
E = 800000
D = 96
L = 16
NCHUNK = D // L
NSUB = 16
NCORE = 2
NPAIR = NCHUNK // NCORE
W = 128
NWIN = 392
EPT = NWIN * W
EPAD = EPT * NSUB
NPAD = 52000
ROWS_PER_SUB = N // NSUB
ZROWS = 125
BN = 2000
NBLK = NPAD // BN


def _node_mm_body(x_ref, w_ref, b_ref, o_ref):
    o_ref[...] = (
        jnp.dot(x_ref[...], w_ref[...], preferred_element_type=jnp.float32)
        + b_ref[...]
    )


def _edge_mm_body(a_ref, w_ref, o_ref):
    o_ref[...] = jnp.dot(a_ref[...], w_ref[...], preferred_element_type=jnp.float32)


def _table_mm_body(x_ref, wd_ref, we_ref, bd_ref, be_ref, dxt_ref, ebt_ref):
    i = pl.program_id(1)

    @pl.when(i < NBLK - 1)
    def _():
        dxt_ref[...] = (
            jnp.dot(x_ref[...], wd_ref[0], preferred_element_type=jnp.float32)
            + bd_ref[0]
        )
        ebt_ref[...] = (
            jnp.dot(x_ref[...], we_ref[0], preferred_element_type=jnp.float32)
            + be_ref[0]
        )

    @pl.when(i == NBLK - 1)
    def _():
        dxt_ref[...] = jnp.full_like(dxt_ref, 1e30)
        ebt_ref[...] = jnp.zeros_like(ebt_ref)


def _final_body(ax_ref, x_ref, g_ref, be_ref, a0, a1, a2, a3, a4, a5,
                o_ref, stat_ref):
    p = pl.program_id(0)
    accs = (a0, a1, a2, a3, a4, a5)
    num = jnp.concatenate([a[...][:, 0:L] for a in accs], axis=1)
    den = jnp.concatenate([a[...][:, L:2 * L] for a in accs], axis=1)
    h = ax_ref[...] + num / (den + 1e-6)

    @pl.when(p == 0)
    def _():
        @pl.when(pl.program_id(1) == 0)
        def _():
            stat_ref[...] = jnp.zeros_like(stat_ref)

        stat_ref[0:1, 0:D] += jnp.sum(h, axis=0, keepdims=True)
        stat_ref[1:2, 0:D] += jnp.sum(h * h, axis=0, keepdims=True)

    @pl.when(p == 1)
    def _():
        mean = stat_ref[0:1, 0:D] / N
        var = stat_ref[1:2, 0:D] / N - mean * mean
        bn = g_ref[...] * (h - mean) / jnp.sqrt(var + 1e-5) + be_ref[...]
        o_ref[...] = x_ref[...] + jnp.maximum(bn, 0.0)


def _sc_body(dxt, ebt, ce, src_idx, dst_idx, out,
             acc, dst_raw, dst_adj, src_adj, scidx, dbuf, ebbuf, cbuf, updbuf,
             sem_i0, sem_i1, sem_g0, sem_g1, sem_s, *, pair):
    core = lax.axis_index("c")
    sid = lax.axis_index("s")
    semi = (sem_i0, sem_i1)
    semg = (sem_g0, sem_g1)
    cbase = core * NPAD
    ccol = (2 * pair) * L + core * L

    def _z(r, carry):
        updbuf[0, r, 0:L] = jnp.zeros((L,), jnp.float32)
        updbuf[0, r, L:2 * L] = jnp.zeros((L,), jnp.float32)
        return carry
    lax.fori_loop(0, ZROWS, _z, 0)

    def _zero(k, carry):
        pltpu.sync_copy(updbuf.at[0, pl.ds(0, ZROWS)],
                        acc.at[pl.ds(sid * ROWS_PER_SUB + k * ZROWS, ZROWS)])
        return carry
    lax.fori_loop(0, ROWS_PER_SUB // ZROWS, _zero, 0)
    plsc.subcore_barrier()

    def issue_i(w, par):
        base = sid * EPT + w * W
        pltpu.async_copy(dst_idx.at[pl.ds(base, W)], dst_raw.at[par], semi[par])
        pltpu.async_copy(src_idx.at[pl.ds(base, W)], src_adj.at[par], semi[par])

    def wait_i(par):
        pltpu.make_async_copy(dst_idx.at[pl.ds(0, W)], dst_raw.at[par], semi[par]).wait()
        pltpu.make_async_copy(src_idx.at[pl.ds(0, W)], src_adj.at[par], semi[par]).wait()

    def do_adj(par):
        for k in range(W // L):
            sl = pl.ds(k * L, L)
            dst_adj[par, sl] = dst_raw[par, sl] + cbase
            src_adj[par, sl] = src_adj[par, sl] + cbase

    def do_clamp(par):
        for k in range(W // L):
            sl = pl.ds(k * L, L)
            scidx[par, sl] = jnp.minimum(dst_raw[par, sl], N - 1)

    def issue_g(w, par):
        base = sid * EPT + w * W
        bce = jnp.minimum(base, E - W)
        pltpu.async_copy(dxt.at[dst_adj.at[par]], dbuf.at[par], semg[par])
        pltpu.async_copy(ebt.at[src_adj.at[par]], ebbuf.at[par], semg[par])
        pltpu.async_copy(ce.at[pl.ds(bce, W), pl.ds(ccol, L)], cbuf.at[par],
                         semg[par])

    def wait_g(par):
        pltpu.make_async_copy(dxt.at[pl.ds(0, W)], dbuf.at[par], semg[par]).wait()
        pltpu.make_async_copy(ebt.at[pl.ds(0, W)], ebbuf.at[par], semg[par]).wait()
        pltpu.make_async_copy(ce.at[pl.ds(0, W), pl.ds(0, L)], cbuf.at[par], semg[par]).wait()

    def wait_s(par):
        pltpu.make_async_copy(ebt.at[pl.ds(0, W)], updbuf.at[par], sem_s).wait()

    def compute(par):
        @plsc.parallel_loop(0, W, unroll=8)
        def _(e):
            ev = dbuf[par, e, :] + ebbuf[par, e, 0:L] + cbuf[par, e, :]
            sig = 1.0 / (1.0 + jnp.exp(ev))
            updbuf[par, e, 0:L] = sig * ebbuf[par, e, L:2 * L]
            updbuf[par, e, L:2 * L] = sig

    def run_window(w, t, par, drain_guard, prep_guard, issue_guard):
        q = 1 - par
        wait_g(par)

        def _prep():
            wait_i(q)
            do_adj(q)
            issue_g(w + 1, q)
        if prep_guard:
            pl.when(t < NWIN // 2 - 1)(_prep)
        else:
            _prep()
        compute(par)

        def _drain():
            wait_s(q)
        if drain_guard:
            pl.when(t > 0)(_drain)
        else:
            _drain()
        do_clamp(par)
        pltpu.async_copy(updbuf.at[par], acc.at[scidx.at[par]], sem_s, add=True)

        def _issue_next():
            issue_i(w + 2, par)
        pl.when(t < NWIN // 2 - 1)(_issue_next)

    issue_i(0, 0)
    issue_i(1, 1)
    wait_i(0)
    do_adj(0)
    issue_g(0, 0)

    def _witer(t, carry):
        run_window(2 * t, t, 0, True, False, True)
        run_window(2 * t + 1, t, 1, False, True, True)
        return carry
    lax.fori_loop(0, NWIN // 2, _witer, 0)
    wait_s(0)
    plsc.subcore_barrier()

    pltpu.sync_copy(
        acc.at[pl.ds(sid * ROWS_PER_SUB, ROWS_PER_SUB)],
        out.at[pl.ds(core * N + sid * ROWS_PER_SUB, ROWS_PER_SUB)],
    )
    plsc.subcore_barrier()


def kernel(x, edge_index, edge_attr, W_A, b_A, W_B, b_B, W_C, b_C, W_D, b_D,
           W_E, b_E, gamma_x, beta_x, gamma_e, beta_e):
    f32 = jnp.float32

    ax = pl.pallas_call(
        _node_mm_body,
        grid=(N // BN,),
        in_specs=[
            pl.BlockSpec((BN, D), lambda i: (i, 0)),
            pl.BlockSpec((D, D), lambda i: (0, 0)),
            pl.BlockSpec((1, D), lambda i: (0, 0)),
        ],
        out_specs=pl.BlockSpec((BN, D), lambda i: (i, 0)),
        out_shape=jax.ShapeDtypeStruct((N, D), f32),
    )(x, W_A, b_A.reshape(1, D))

    npd = EPAD - E
    ar = jnp.arange(npd, dtype=jnp.int32)
    src_pad = jnp.concatenate([edge_index[0], ar % 64])
    dst_pad = jnp.concatenate([edge_index[1], N + (ar % 64)])

    wd4 = (-W_D).reshape(D, NCHUNK, L).transpose(1, 0, 2)
    we4 = jnp.concatenate(
        [(-W_E).reshape(D, NCHUNK, L), W_B.reshape(D, NCHUNK, L)], axis=2
    ).transpose(1, 0, 2)
    bd4 = (-(b_D + b_E + b_C)).reshape(NCHUNK, 1, L)
    be4 = jnp.concatenate(
        [jnp.zeros((NCHUNK, 1, L), f32), b_B.reshape(NCHUNK, 1, L)], axis=2)
    wc128 = jnp.concatenate([-W_C, jnp.zeros((D, 2 * L), f32)], axis=1)

    mesh = plsc.VectorSubcoreMesh(core_axis_name="c", subcore_axis_name="s")
    BE = 2000

    ce = pl.pallas_call(
        _edge_mm_body,
        grid=(E // BE,),
        in_specs=[
            pl.BlockSpec((BE, D), lambda i: (i, 0)),
            pl.BlockSpec((D, 128), lambda i: (0, 0)),
        ],
        out_specs=pl.BlockSpec((BE, 128), lambda i: (i, 0)),
        out_shape=jax.ShapeDtypeStruct((E, 128), f32),
    )(edge_attr, wc128)

    accs = []
    for j in range(NPAIR):
        dxt, ebt = pl.pallas_call(
            _table_mm_body,
            grid=(NCORE, NBLK),
            in_specs=[
                pl.BlockSpec((BN, D), lambda c, i: (jnp.minimum(i, NBLK - 2), 0)),
                pl.BlockSpec((1, D, L), lambda c, i: (c, 0, 0)),
                pl.BlockSpec((1, D, 2 * L), lambda c, i: (c, 0, 0)),
                pl.BlockSpec((1, 1, L), lambda c, i: (c, 0, 0)),
                pl.BlockSpec((1, 1, 2 * L), lambda c, i: (c, 0, 0)),
            ],
            out_specs=[
                pl.BlockSpec((BN, L), lambda c, i: (c * NBLK + i, 0)),
                pl.BlockSpec((BN, 2 * L), lambda c, i: (c * NBLK + i, 0)),
            ],
            out_shape=[
                jax.ShapeDtypeStruct((NCORE * NPAD, L), f32),
                jax.ShapeDtypeStruct((NCORE * NPAD, 2 * L), f32),
            ],
        )(x, wd4[2 * j:2 * j + 2], we4[2 * j:2 * j + 2],
          bd4[2 * j:2 * j + 2], be4[2 * j:2 * j + 2])

        accs.append(pl.kernel(
            functools.partial(_sc_body, pair=j),
            out_type=jax.ShapeDtypeStruct((NCORE * N, 2 * L), f32),
            mesh=mesh,
            compiler_params=pltpu.CompilerParams(
                use_tc_tiling_on_sc=False, internal_scratch_in_bytes=256 * 1024),
            scratch_types=[
                pltpu.VMEM_SHARED((N, 2 * L), f32),
                pltpu.VMEM((2, W), jnp.int32),
                pltpu.VMEM((2, W), jnp.int32),
                pltpu.VMEM((2, W), jnp.int32),
                pltpu.VMEM((2, W), jnp.int32),
                pltpu.VMEM((2, W, L), f32),
                pltpu.VMEM((2, W, 2 * L), f32),
                pltpu.VMEM((2, W, L), f32),
                pltpu.VMEM((2, W, 2 * L), f32),
                pltpu.SemaphoreType.DMA,
                pltpu.SemaphoreType.DMA,
                pltpu.SemaphoreType.DMA,
                pltpu.SemaphoreType.DMA,
                pltpu.SemaphoreType.DMA,
            ],
        )(dxt, ebt, ce, src_pad, dst_pad))

    def _accspec(c):
        return pl.BlockSpec((BN, 2 * L), lambda p, i, c=c: (c * (N // BN) + i, 0))

    x_out = pl.pallas_call(
        _final_body,
        grid=(2, N // BN),
        in_specs=[
            pl.BlockSpec((BN, D), lambda p, i: (i, 0)),
            pl.BlockSpec((BN, D), lambda p, i: (i, 0)),
            pl.BlockSpec((1, D), lambda p, i: (0, 0)),
            pl.BlockSpec((1, D), lambda p, i: (0, 0)),
        ] + [_accspec(c) for _ in range(NPAIR) for c in range(NCORE)],
        out_specs=pl.BlockSpec((BN, D), lambda p, i: (i, 0)),
        out_shape=jax.ShapeDtypeStruct((N, D), f32),
        scratch_shapes=[pltpu.VMEM((8, 128), f32)],
    )(ax, x, gamma_x.reshape(1, D), beta_x.reshape(1, D),
      accs[0], accs[0], accs[1], accs[1], accs[2], accs[2])

    return x_out
